# baseline (speedup 1.0000x reference)
import functools

import jax
import jax.numpy as jnp
from jax import lax
from jax.experimental import pallas as pl
from jax.experimental.pallas import tpu as pltpu
from jax.experimental.pallas import tpu_sc as plsc

_N = 2048
_NS = 16
_L = 16
_PER = _N // _NS
_NV = _PER // _L
_STEPS = 7
_LO = (1.0 - 1e-06) ** 2
_HI = (1.0 + 1e-06) ** 2


def _full(v, dtype=jnp.float32):
    return jnp.full((_L,), v, dtype)


def _rsqrt16(x):
    i = lax.bitcast_convert_type(x, jnp.int32)
    g = lax.bitcast_convert_type(
        jnp.full((_L,), 0x5F3759DF, jnp.int32) - (i >> 1), jnp.float32)
    for _ in range(4):
        g = g * (1.5 - 0.5 * x * g * g)
    return g


def _sqrt16(x):
    return x * _rsqrt16(jnp.maximum(x, 1e-37))


def _bsum(v):
    return _full(jnp.sum(v))


def _kabsch16(S, asum, msum):
    inv_n = 1.0 / _N
    mu1 = [asum[k] * inv_n for k in range(3)]
    mu2 = [msum[k] * inv_n for k in range(3)]
    H = [[S[i][j] - asum[i] * msum[j] * inv_n for j in range(3)]
         for i in range(3)]

    one = _full(1.0)
    zero = _full(0.0)
    K = [[H[0][i] * H[0][j] + H[1][i] * H[1][j] + H[2][i] * H[2][j]
          for j in range(3)] for i in range(3)]
    V = [[one if i == j else zero for j in range(3)] for i in range(3)]

    for _ in range(4):
        for (p, q) in ((0, 1), (0, 2), (1, 2)):
            apq = K[p][q]
            small = jnp.abs(apq) < 1e-30
            apq_safe = jnp.where(small, one, apq)
            tau = (K[q][q] - K[p][p]) * 0.5 / apq_safe
            sgn = jnp.where(tau >= 0.0, one, -one)
            tt = sgn / (jnp.abs(tau) + _sqrt16(1.0 + tau * tau))
            t_ = jnp.where(small, zero, tt)
            c = _rsqrt16(1.0 + t_ * t_)
            s = t_ * c
            Kn = [row[:] for row in K]
            kpp = K[p][p]
            kqq = K[q][q]
            Kn[p][p] = c * c * kpp - 2.0 * s * c * apq + s * s * kqq
            Kn[q][q] = s * s * kpp + 2.0 * s * c * apq + c * c * kqq
            Kn[p][q] = zero
            Kn[q][p] = zero
            r = 3 - p - q
            krp = K[r][p]
            krq = K[r][q]
            Kn[r][p] = c * krp - s * krq
            Kn[p][r] = Kn[r][p]
            Kn[r][q] = s * krp + c * krq
            Kn[q][r] = Kn[r][q]
            K = Kn
            Vn = [row[:] for row in V]
            for rr in range(3):
                vrp = V[rr][p]
                vrq = V[rr][q]
                Vn[rr][p] = c * vrp - s * vrq
                Vn[rr][q] = s * vrp + c * vrq
            V = Vn

    e = [K[0][0], K[1][1], K[2][2]]

    def cswap(e, V, i, j):
        sw = e[i] < e[j]
        e2 = e[:]
        e2[i] = jnp.where(sw, e[j], e[i])
        e2[j] = jnp.where(sw, e[i], e[j])
        V2 = [row[:] for row in V]
        for r in range(3):
            V2[r][i] = jnp.where(sw, V[r][j], V[r][i])
            V2[r][j] = jnp.where(sw, V[r][i], V[r][j])
        return e2, V2

    e, V = cswap(e, V, 0, 1)
    e, V = cswap(e, V, 1, 2)
    e, V = cswap(e, V, 0, 1)

    rs = [_rsqrt16(jnp.maximum(e[i], 1e-30)) for i in range(3)]
    U = [[(H[m][0] * V[0][i] + H[m][1] * V[1][i] + H[m][2] * V[2][i]) * rs[i]
          for i in range(3)] for m in range(3)]
    detH = (H[0][0] * (H[1][1] * H[2][2] - H[1][2] * H[2][1])
            - H[0][1] * (H[1][0] * H[2][2] - H[1][2] * H[2][0])
            + H[0][2] * (H[1][0] * H[2][1] - H[1][1] * H[2][0]))
    d = jnp.where(detH >= 0.0, one, -one)
    dd = [one, one, d]
    R = [[dd[0] * V[a][0] * U[b][0] + dd[1] * V[a][1] * U[b][1]
          + dd[2] * V[a][2] * U[b][2] for b in range(3)] for a in range(3)]
    t = [mu2[a] - (R[a][0] * mu1[0] + R[a][1] * mu1[1] + R[a][2] * mu1[2])
         for a in range(3)]
    return R, t


def _icp_sc_body(p1x, p1y, p1z, p2x, p2y, p2z, out_hbm,
                 p2xr, p2yr, p2zr, nbxr, nbyr, nbzr, bbr,
                 oxr, oyr, ozr, sxr, syr, szr, d2or,
                 partials, allsums, outv, shared):
    sid = lax.axis_index("s")
    cid = lax.axis_index("c")
    base = sid * _PER

    pltpu.sync_copy(p2x, p2xr)
    pltpu.sync_copy(p2y, p2yr)
    pltpu.sync_copy(p2z, p2zr)
    pltpu.sync_copy(p1x.at[pl.ds(base, _PER)], oxr)
    pltpu.sync_copy(p1y.at[pl.ds(base, _PER)], oyr)
    pltpu.sync_copy(p1z.at[pl.ds(base, _PER)], ozr)
    pltpu.sync_copy(p1x.at[pl.ds(base, _PER)], sxr)
    pltpu.sync_copy(p1y.at[pl.ds(base, _PER)], syr)
    pltpu.sync_copy(p1z.at[pl.ds(base, _PER)], szr)

    def prep(k, _):
        sl = pl.ds(k * _L, _L)
        vx = p2xr[sl]
        vy = p2yr[sl]
        vz = p2zr[sl]
        nbxr[sl] = -2.0 * vx
        nbyr[sl] = -2.0 * vy
        nbzr[sl] = -2.0 * vz
        bbr[sl] = vx * vx + vy * vy + vz * vz
        return 0

    lax.fori_loop(0, _N // _L, prep, 0)

    def reduce_shared():
        pltpu.sync_copy(partials, shared.at[pl.ds(sid * 256, 256)])
        plsc.subcore_barrier()
        pltpu.sync_copy(shared, allsums)
        vecs, scals = [], []
        for k in range(16):
            acc = allsums[pl.ds(k * _L, _L)]
            for s in range(1, _NS):
                acc = acc + allsums[pl.ds(s * 256 + k * _L, _L)]
            sc_ = jnp.sum(acc)
            vecs.append(_full(sc_))
            scals.append(sc_)
        return vecs, scals

    def do_step(step):
        sx = [sxr[pl.ds(v * _L, _L)] for v in range(_NV)]
        sy = [syr[pl.ds(v * _L, _L)] for v in range(_NV)]
        sz = [szr[pl.ds(v * _L, _L)] for v in range(_NV)]

        def jbody(jc, carry):
            smins, cmins = carry
            j0 = jc * _L
            sl = pl.ds(j0, _L)
            cxv = nbxr[sl]
            cyv = nbyr[sl]
            czv = nbzr[sl]
            cbv = bbr[sl]
            sprev = list(smins)
            smins = list(smins)
            cmins = list(cmins)
            for k in range(_L):
                cx = cxv[k]
                cy = cyv[k]
                cz = czv[k]
                cb = cbv[k]
                for v in range(_NV):
                    sc_ = sx[v] * cx + sy[v] * cy + sz[v] * cz + cb
                    smins[v] = jnp.minimum(smins[v], sc_)
            jcv = jnp.full((_L,), jc, jnp.int32)
            for v in range(_NV):
                chg = smins[v] != sprev[v]
                cmins[v] = jnp.where(chg, jcv, cmins[v])
            return (tuple(smins), tuple(cmins))

        init = (tuple(_full(3.0e38) for _ in range(_NV)),
                tuple(jnp.zeros((_L,), jnp.int32) for _ in range(_NV)))
        smins, cmins = lax.fori_loop(0, _N // _L, jbody, init, unroll=4)

        big = jnp.full((_L,), 1 << 20, jnp.int32)
        jmins = []
        for v in range(_NV):
            jbase = cmins[v] * _L
            found = big
            for k in range(_L):
                jv = jbase + k
                cx = plsc.load_gather(nbxr, [jv])
                cy = plsc.load_gather(nbyr, [jv])
                cz = plsc.load_gather(nbzr, [jv])
                cb = plsc.load_gather(bbr, [jv])
                sc_ = sx[v] * cx + sy[v] * cy + sz[v] * cz + cb
                eq = sc_ == smins[v]
                found = jnp.minimum(found, jnp.where(eq, jv, big))
            jmins.append(jnp.minimum(found, _N - 1))

        acc = [_full(0.0) for _ in range(16)]
        bad = _full(0.0)
        for v in range(_NV):
            mx = plsc.load_gather(p2xr, [jmins[v]])
            my = plsc.load_gather(p2yr, [jmins[v]])
            mz = plsc.load_gather(p2zr, [jmins[v]])
            acc[0] += sx[v] * mx
            acc[1] += sx[v] * my
            acc[2] += sx[v] * mz
            acc[3] += sy[v] * mx
            acc[4] += sy[v] * my
            acc[5] += sy[v] * mz
            acc[6] += sz[v] * mx
            acc[7] += sz[v] * my
            acc[8] += sz[v] * mz
            acc[9] += mx
            acc[10] += my
            acc[11] += mz
            acc[12] += sx[v]
            acc[13] += sy[v]
            acc[14] += sz[v]
            d2 = smins[v] + sx[v] * sx[v] + sy[v] * sy[v] + sz[v] * sz[v]
            d2 = jnp.maximum(d2, 0.0)
            sl = pl.ds(v * _L, _L)
            d2o = d2or[sl]
            good = jnp.logical_and(d2 > _LO * d2o, d2 < _HI * d2o)
            bad = bad + jnp.where(good, _full(0.0), _full(1.0))
            d2or[sl] = d2
        acc[15] = bad

        for k in range(16):
            partials[pl.ds(k * _L, _L)] = acc[k]
        sums, scals = reduce_shared()
        plsc.subcore_barrier()

        S = [[sums[3 * i + j] for j in range(3)] for i in range(3)]
        msum = [sums[9], sums[10], sums[11]]
        asum = [sums[12], sums[13], sums[14]]
        badtot = scals[15]
        R, t = _kabsch16(S, asum, msum)

        for v in range(_NV):
            sl = pl.ds(v * _L, _L)
            nx = R[0][0] * sx[v] + R[0][1] * sy[v] + R[0][2] * sz[v] + t[0]
            ny = R[1][0] * sx[v] + R[1][1] * sy[v] + R[1][2] * sz[v] + t[1]
            nz = R[2][0] * sx[v] + R[2][1] * sy[v] + R[2][2] * sz[v] + t[2]
            sxr[sl] = nx
            syr[sl] = ny
            szr[sl] = nz

        first = step == 0
        conv = jnp.logical_and(jnp.logical_not(first), badtot == 0.0)
        return jnp.where(conv, 1, 0).astype(jnp.int32)

    def loop_body(step, conv):
        return lax.cond(conv == 0, lambda: do_step(step), lambda: conv)

    lax.fori_loop(0, _STEPS, loop_body, jnp.asarray(0, jnp.int32))

    acc = [_full(0.0) for _ in range(16)]
    for v in range(_NV):
        sl = pl.ds(v * _L, _L)
        ox = oxr[sl]
        oy = oyr[sl]
        oz = ozr[sl]
        mx = sxr[sl]
        my = syr[sl]
        mz = szr[sl]
        acc[0] += ox * mx
        acc[1] += ox * my
        acc[2] += ox * mz
        acc[3] += oy * mx
        acc[4] += oy * my
        acc[5] += oy * mz
        acc[6] += oz * mx
        acc[7] += oz * my
        acc[8] += oz * mz
        acc[9] += mx
        acc[10] += my
        acc[11] += mz
        acc[12] += ox
        acc[13] += oy
        acc[14] += oz
    for k in range(16):
        partials[pl.ds(k * _L, _L)] = acc[k]
    sums, _scals = reduce_shared()

    S = [[sums[3 * i + j] for j in range(3)] for i in range(3)]
    msum = [sums[9], sums[10], sums[11]]
    asum = [sums[12], sums[13], sums[14]]
    R, t = _kabsch16(S, asum, msum)

    @pl.when(jnp.logical_and(sid == 0, cid == 0))
    def _():
        lane = lax.iota(jnp.int32, _L)
        ov = _full(0.0)
        vals = [R[0][0], R[0][1], R[0][2], t[0],
                R[1][0], R[1][1], R[1][2], t[1],
                R[2][0], R[2][1], R[2][2], t[2]]
        for k, val in enumerate(vals):
            ov = jnp.where(lane == k, val, ov)
        outv[...] = ov
        pltpu.sync_copy(outv, out_hbm)


@functools.partial(jax.jit, static_argnums=())
def _icp_sc(p1x, p1y, p1z, p2x, p2y, p2z):
    f32 = jnp.float32
    mesh = plsc.VectorSubcoreMesh(core_axis_name="c", subcore_axis_name="s",
                                  num_cores=2, num_subcores=_NS)
    scratch = [
        pltpu.VMEM((_N,), f32), pltpu.VMEM((_N,), f32), pltpu.VMEM((_N,), f32),
        pltpu.VMEM((_N,), f32), pltpu.VMEM((_N,), f32), pltpu.VMEM((_N,), f32),
        pltpu.VMEM((_N,), f32),
        pltpu.VMEM((_PER,), f32), pltpu.VMEM((_PER,), f32), pltpu.VMEM((_PER,), f32),
        pltpu.VMEM((_PER,), f32), pltpu.VMEM((_PER,), f32), pltpu.VMEM((_PER,), f32),
        pltpu.VMEM((_PER,), f32),
        pltpu.VMEM((256,), f32),
        pltpu.VMEM((_NS * 256,), f32),
        pltpu.VMEM((_L,), f32),
        pltpu.VMEM_SHARED((_NS * 256,), f32),
    ]
    return pl.kernel(
        _icp_sc_body,
        out_type=jax.ShapeDtypeStruct((_L,), f32),
        mesh=mesh,
        scratch_types=scratch,
        compiler_params=pltpu.CompilerParams(needs_layout_passes=False),
    )(p1x, p1y, p1z, p2x, p2y, p2z)


def kernel(p1, p2):
    a = p1[0]
    b = p2[0]
    out16 = _icp_sc(a[:, 0], a[:, 1], a[:, 2], b[:, 0], b[:, 1], b[:, 2])
    return out16[:12].reshape(1, 3, 4)


# speedup vs baseline: 1.0190x; 1.0190x over previous
import functools

import jax
import jax.numpy as jnp
from jax import lax
from jax.experimental import pallas as pl
from jax.experimental.pallas import tpu as pltpu
from jax.experimental.pallas import tpu_sc as plsc

_N = 2048
_NS = 16
_L = 16
_PER = _N // _NS
_NV = _PER // _L
_STEPS = 7
_LO = (1.0 - 1e-06) ** 2
_HI = (1.0 + 1e-06) ** 2


def _full(v, dtype=jnp.float32):
    return jnp.full((_L,), v, dtype)


def _rsqrt16(x):
    i = lax.bitcast_convert_type(x, jnp.int32)
    g = lax.bitcast_convert_type(
        jnp.full((_L,), 0x5F3759DF, jnp.int32) - (i >> 1), jnp.float32)
    for _ in range(4):
        g = g * (1.5 - 0.5 * x * g * g)
    return g


def _sqrt16(x):
    return x * _rsqrt16(jnp.maximum(x, 1e-37))


def _bsum(v):
    return _full(jnp.sum(v))


def _kabsch16(S, asum, msum):
    inv_n = 1.0 / _N
    mu1 = [asum[k] * inv_n for k in range(3)]
    mu2 = [msum[k] * inv_n for k in range(3)]
    H = [[S[i][j] - asum[i] * msum[j] * inv_n for j in range(3)]
         for i in range(3)]

    one = _full(1.0)
    zero = _full(0.0)
    K = [[H[0][i] * H[0][j] + H[1][i] * H[1][j] + H[2][i] * H[2][j]
          for j in range(3)] for i in range(3)]
    V = [[one if i == j else zero for j in range(3)] for i in range(3)]

    for _ in range(4):
        for (p, q) in ((0, 1), (0, 2), (1, 2)):
            apq = K[p][q]
            small = jnp.abs(apq) < 1e-30
            apq_safe = jnp.where(small, one, apq)
            tau = (K[q][q] - K[p][p]) * 0.5 / apq_safe
            sgn = jnp.where(tau >= 0.0, one, -one)
            tt = sgn / (jnp.abs(tau) + _sqrt16(1.0 + tau * tau))
            t_ = jnp.where(small, zero, tt)
            c = _rsqrt16(1.0 + t_ * t_)
            s = t_ * c
            Kn = [row[:] for row in K]
            kpp = K[p][p]
            kqq = K[q][q]
            Kn[p][p] = c * c * kpp - 2.0 * s * c * apq + s * s * kqq
            Kn[q][q] = s * s * kpp + 2.0 * s * c * apq + c * c * kqq
            Kn[p][q] = zero
            Kn[q][p] = zero
            r = 3 - p - q
            krp = K[r][p]
            krq = K[r][q]
            Kn[r][p] = c * krp - s * krq
            Kn[p][r] = Kn[r][p]
            Kn[r][q] = s * krp + c * krq
            Kn[q][r] = Kn[r][q]
            K = Kn
            Vn = [row[:] for row in V]
            for rr in range(3):
                vrp = V[rr][p]
                vrq = V[rr][q]
                Vn[rr][p] = c * vrp - s * vrq
                Vn[rr][q] = s * vrp + c * vrq
            V = Vn

    e = [K[0][0], K[1][1], K[2][2]]

    def cswap(e, V, i, j):
        sw = e[i] < e[j]
        e2 = e[:]
        e2[i] = jnp.where(sw, e[j], e[i])
        e2[j] = jnp.where(sw, e[i], e[j])
        V2 = [row[:] for row in V]
        for r in range(3):
            V2[r][i] = jnp.where(sw, V[r][j], V[r][i])
            V2[r][j] = jnp.where(sw, V[r][i], V[r][j])
        return e2, V2

    e, V = cswap(e, V, 0, 1)
    e, V = cswap(e, V, 1, 2)
    e, V = cswap(e, V, 0, 1)

    rs = [_rsqrt16(jnp.maximum(e[i], 1e-30)) for i in range(3)]
    U = [[(H[m][0] * V[0][i] + H[m][1] * V[1][i] + H[m][2] * V[2][i]) * rs[i]
          for i in range(3)] for m in range(3)]
    detH = (H[0][0] * (H[1][1] * H[2][2] - H[1][2] * H[2][1])
            - H[0][1] * (H[1][0] * H[2][2] - H[1][2] * H[2][0])
            + H[0][2] * (H[1][0] * H[2][1] - H[1][1] * H[2][0]))
    d = jnp.where(detH >= 0.0, one, -one)
    dd = [one, one, d]
    R = [[dd[0] * V[a][0] * U[b][0] + dd[1] * V[a][1] * U[b][1]
          + dd[2] * V[a][2] * U[b][2] for b in range(3)] for a in range(3)]
    t = [mu2[a] - (R[a][0] * mu1[0] + R[a][1] * mu1[1] + R[a][2] * mu1[2])
         for a in range(3)]
    return R, t


def _icp_sc_body(p1x, p1y, p1z, p2x, p2y, p2z, out_hbm,
                 p2xr, p2yr, p2zr, nbxr, nbyr, nbzr, bbr,
                 oxr, oyr, ozr, sxr, syr, szr, d2or,
                 partials, allsums, outv, shared):
    sid = lax.axis_index("s")
    cid = lax.axis_index("c")
    base = sid * _PER

    pltpu.sync_copy(p2x, p2xr)
    pltpu.sync_copy(p2y, p2yr)
    pltpu.sync_copy(p2z, p2zr)
    pltpu.sync_copy(p1x.at[pl.ds(base, _PER)], oxr)
    pltpu.sync_copy(p1y.at[pl.ds(base, _PER)], oyr)
    pltpu.sync_copy(p1z.at[pl.ds(base, _PER)], ozr)
    pltpu.sync_copy(p1x.at[pl.ds(base, _PER)], sxr)
    pltpu.sync_copy(p1y.at[pl.ds(base, _PER)], syr)
    pltpu.sync_copy(p1z.at[pl.ds(base, _PER)], szr)

    def prep(k, _):
        sl = pl.ds(k * _L, _L)
        vx = p2xr[sl]
        vy = p2yr[sl]
        vz = p2zr[sl]
        nbxr[sl] = -2.0 * vx
        nbyr[sl] = -2.0 * vy
        nbzr[sl] = -2.0 * vz
        bbr[sl] = vx * vx + vy * vy + vz * vz
        return 0

    lax.fori_loop(0, _N // _L, prep, 0)

    def reduce_shared():
        pltpu.sync_copy(partials, shared.at[pl.ds(sid * 256, 256)])
        plsc.subcore_barrier()
        pltpu.sync_copy(shared, allsums)
        vecs, scals = [], []
        for k in range(16):
            acc = allsums[pl.ds(k * _L, _L)]
            for s in range(1, _NS):
                acc = acc + allsums[pl.ds(s * 256 + k * _L, _L)]
            sc_ = jnp.sum(acc)
            vecs.append(_full(sc_))
            scals.append(sc_)
        return vecs, scals

    def do_step(step):
        sx = [sxr[pl.ds(v * _L, _L)] for v in range(_NV)]
        sy = [syr[pl.ds(v * _L, _L)] for v in range(_NV)]
        sz = [szr[pl.ds(v * _L, _L)] for v in range(_NV)]

        def jbody(jc, carry):
            smins, cmins = carry
            j0 = jc * _L
            sl = pl.ds(j0, _L)
            cxv = nbxr[sl]
            cyv = nbyr[sl]
            czv = nbzr[sl]
            cbv = bbr[sl]
            sprev = list(smins)
            smins = list(smins)
            cmins = list(cmins)
            for k in range(_L):
                cx = cxv[k]
                cy = cyv[k]
                cz = czv[k]
                cb = cbv[k]
                for v in range(_NV):
                    sc_ = sx[v] * cx + sy[v] * cy + sz[v] * cz + cb
                    smins[v] = jnp.minimum(smins[v], sc_)
            jcv = jnp.full((_L,), jc, jnp.int32)
            for v in range(_NV):
                chg = smins[v] != sprev[v]
                cmins[v] = jnp.where(chg, jcv, cmins[v])
            return (tuple(smins), tuple(cmins))

        init = (tuple(_full(3.0e38) for _ in range(_NV)),
                tuple(jnp.zeros((_L,), jnp.int32) for _ in range(_NV)))
        smins, cmins = lax.fori_loop(0, _N // _L, jbody, init, unroll=2)

        big = jnp.full((_L,), 1 << 20, jnp.int32)
        jmins = []
        for v in range(_NV):
            jbase = cmins[v] * _L
            found = big
            for k in range(_L):
                jv = jbase + k
                cx = plsc.load_gather(nbxr, [jv])
                cy = plsc.load_gather(nbyr, [jv])
                cz = plsc.load_gather(nbzr, [jv])
                cb = plsc.load_gather(bbr, [jv])
                sc_ = sx[v] * cx + sy[v] * cy + sz[v] * cz + cb
                eq = sc_ == smins[v]
                found = jnp.minimum(found, jnp.where(eq, jv, big))
            jmins.append(jnp.minimum(found, _N - 1))

        acc = [_full(0.0) for _ in range(16)]
        bad = _full(0.0)
        for v in range(_NV):
            mx = plsc.load_gather(p2xr, [jmins[v]])
            my = plsc.load_gather(p2yr, [jmins[v]])
            mz = plsc.load_gather(p2zr, [jmins[v]])
            acc[0] += sx[v] * mx
            acc[1] += sx[v] * my
            acc[2] += sx[v] * mz
            acc[3] += sy[v] * mx
            acc[4] += sy[v] * my
            acc[5] += sy[v] * mz
            acc[6] += sz[v] * mx
            acc[7] += sz[v] * my
            acc[8] += sz[v] * mz
            acc[9] += mx
            acc[10] += my
            acc[11] += mz
            acc[12] += sx[v]
            acc[13] += sy[v]
            acc[14] += sz[v]
            d2 = smins[v] + sx[v] * sx[v] + sy[v] * sy[v] + sz[v] * sz[v]
            d2 = jnp.maximum(d2, 0.0)
            sl = pl.ds(v * _L, _L)
            d2o = d2or[sl]
            good = jnp.logical_and(d2 > _LO * d2o, d2 < _HI * d2o)
            bad = bad + jnp.where(good, _full(0.0), _full(1.0))
            d2or[sl] = d2
        acc[15] = bad

        for k in range(16):
            partials[pl.ds(k * _L, _L)] = acc[k]
        sums, scals = reduce_shared()
        plsc.subcore_barrier()

        S = [[sums[3 * i + j] for j in range(3)] for i in range(3)]
        msum = [sums[9], sums[10], sums[11]]
        asum = [sums[12], sums[13], sums[14]]
        badtot = scals[15]
        R, t = _kabsch16(S, asum, msum)

        for v in range(_NV):
            sl = pl.ds(v * _L, _L)
            nx = R[0][0] * sx[v] + R[0][1] * sy[v] + R[0][2] * sz[v] + t[0]
            ny = R[1][0] * sx[v] + R[1][1] * sy[v] + R[1][2] * sz[v] + t[1]
            nz = R[2][0] * sx[v] + R[2][1] * sy[v] + R[2][2] * sz[v] + t[2]
            sxr[sl] = nx
            syr[sl] = ny
            szr[sl] = nz

        first = step == 0
        conv = jnp.logical_and(jnp.logical_not(first), badtot == 0.0)
        return jnp.where(conv, 1, 0).astype(jnp.int32)

    def loop_body(step, conv):
        return lax.cond(conv == 0, lambda: do_step(step), lambda: conv)

    lax.fori_loop(0, _STEPS, loop_body, jnp.asarray(0, jnp.int32))

    acc = [_full(0.0) for _ in range(16)]
    for v in range(_NV):
        sl = pl.ds(v * _L, _L)
        ox = oxr[sl]
        oy = oyr[sl]
        oz = ozr[sl]
        mx = sxr[sl]
        my = syr[sl]
        mz = szr[sl]
        acc[0] += ox * mx
        acc[1] += ox * my
        acc[2] += ox * mz
        acc[3] += oy * mx
        acc[4] += oy * my
        acc[5] += oy * mz
        acc[6] += oz * mx
        acc[7] += oz * my
        acc[8] += oz * mz
        acc[9] += mx
        acc[10] += my
        acc[11] += mz
        acc[12] += ox
        acc[13] += oy
        acc[14] += oz
    for k in range(16):
        partials[pl.ds(k * _L, _L)] = acc[k]
    sums, _scals = reduce_shared()

    S = [[sums[3 * i + j] for j in range(3)] for i in range(3)]
    msum = [sums[9], sums[10], sums[11]]
    asum = [sums[12], sums[13], sums[14]]
    R, t = _kabsch16(S, asum, msum)

    @pl.when(jnp.logical_and(sid == 0, cid == 0))
    def _():
        lane = lax.iota(jnp.int32, _L)
        ov = _full(0.0)
        vals = [R[0][0], R[0][1], R[0][2], t[0],
                R[1][0], R[1][1], R[1][2], t[1],
                R[2][0], R[2][1], R[2][2], t[2]]
        for k, val in enumerate(vals):
            ov = jnp.where(lane == k, val, ov)
        outv[...] = ov
        pltpu.sync_copy(outv, out_hbm)


@functools.partial(jax.jit, static_argnums=())
def _icp_sc(p1x, p1y, p1z, p2x, p2y, p2z):
    f32 = jnp.float32
    mesh = plsc.VectorSubcoreMesh(core_axis_name="c", subcore_axis_name="s",
                                  num_cores=2, num_subcores=_NS)
    scratch = [
        pltpu.VMEM((_N,), f32), pltpu.VMEM((_N,), f32), pltpu.VMEM((_N,), f32),
        pltpu.VMEM((_N,), f32), pltpu.VMEM((_N,), f32), pltpu.VMEM((_N,), f32),
        pltpu.VMEM((_N,), f32),
        pltpu.VMEM((_PER,), f32), pltpu.VMEM((_PER,), f32), pltpu.VMEM((_PER,), f32),
        pltpu.VMEM((_PER,), f32), pltpu.VMEM((_PER,), f32), pltpu.VMEM((_PER,), f32),
        pltpu.VMEM((_PER,), f32),
        pltpu.VMEM((256,), f32),
        pltpu.VMEM((_NS * 256,), f32),
        pltpu.VMEM((_L,), f32),
        pltpu.VMEM_SHARED((_NS * 256,), f32),
    ]
    return pl.kernel(
        _icp_sc_body,
        out_type=jax.ShapeDtypeStruct((_L,), f32),
        mesh=mesh,
        scratch_types=scratch,
        compiler_params=pltpu.CompilerParams(needs_layout_passes=False),
    )(p1x, p1y, p1z, p2x, p2y, p2z)


def kernel(p1, p2):
    a = p1[0]
    b = p2[0]
    out16 = _icp_sc(a[:, 0], a[:, 1], a[:, 2], b[:, 0], b[:, 1], b[:, 2])
    return out16[:12].reshape(1, 3, 4)
